# baseline (speedup 1.0000x reference)
import functools

import jax
import jax.numpy as jnp
from jax import lax
from jax.experimental import pallas as pl
from jax.experimental.pallas import tpu as pltpu
from jax.experimental.pallas import tpu_sc as plsc

N = 50000
E = 1600000
NUM_GRAPHS = 512
HPAD = 8
BE = 4000
RP = 12544
EP = RP * 128
NPAD = 51200
NROWS = NPAD // 128
ROWS_G = RP // 32
ROWS_S = RP // 16
KG = 8
KB = 16
NR_W = NROWS // 16
GPAD = 640

_MESH = plsc.VectorSubcoreMesh(core_axis_name="c", subcore_axis_name="s")


def _gather_phase(acc, dsti, srci, xi_out, xj_out,
                  idxd, idxs, rowsi, rowsj, semi, semj):
    wid = lax.axis_index("s") * 2 + lax.axis_index("c")
    base = wid * ROWS_G

    def chunk(g, carry):
        row0 = base + g * KG
        pltpu.sync_copy(dsti.at[pl.ds(row0, KG)], idxd)
        pltpu.sync_copy(srci.at[pl.ds(row0, KG)], idxs)
        cps = []
        for b in range(KG):
            cps.append(pltpu.async_copy(acc.at[idxd.at[b]], rowsi.at[b], semi))
            cps.append(pltpu.async_copy(acc.at[idxs.at[b]], rowsj.at[b], semj))
        for c in cps:
            c.wait()
        pltpu.sync_copy(rowsi, xi_out.at[pl.ds(row0, KG)])
        pltpu.sync_copy(rowsj, xj_out.at[pl.ds(row0, KG)])
        return carry

    lax.fori_loop(0, ROWS_G // KG, chunk, 0)


def _scatter_phase(acc, msg3, dsti, zeros, idxb, rowsb, sem):
    w = lax.axis_index("s")
    nbase = w * (NPAD // 16)
    pltpu.sync_copy(zeros.at[pl.ds(nbase, NPAD // 16)],
                    acc.at[pl.ds(nbase, NPAD // 16)])
    plsc.subcore_barrier()

    def chunk(g, carry):
        row0 = w * ROWS_S + g * KB
        pltpu.sync_copy(dsti.at[pl.ds(row0, KB)], idxb)
        pltpu.sync_copy(msg3.at[pl.ds(row0, KB)], rowsb)
        cps = [pltpu.async_copy(rowsb.at[b], acc.at[idxb.at[b]], sem,
                                add=True)
               for b in range(KB)]
        for c in cps:
            c.wait()
        return carry

    lax.fori_loop(0, ROWS_S // KB, chunk, 0)
    plsc.subcore_barrier()


@functools.partial(
    pl.kernel,
    out_type=(jax.ShapeDtypeStruct((RP, 128, HPAD), jnp.float32),
              jax.ShapeDtypeStruct((RP, 128, HPAD), jnp.float32)),
    mesh=_MESH,
    compiler_params=pltpu.CompilerParams(use_tc_tiling_on_sc=False),
    scratch_types=[
        pltpu.VMEM_SHARED((NPAD, HPAD), jnp.float32),
        pltpu.VMEM((KG, 128), jnp.int32),
        pltpu.VMEM((KG, 128), jnp.int32),
        pltpu.VMEM((KG, 128, HPAD), jnp.float32),
        pltpu.VMEM((KG, 128, HPAD), jnp.float32),
        pltpu.SemaphoreType.DMA,
        pltpu.SemaphoreType.DMA,
    ],
)
def _sc_gather0(h8, dsti, srci, xi_out, xj_out,
                acc, idxd, idxs, rowsi, rowsj, semi, semj):
    w = lax.axis_index("s")
    nbase = w * (NPAD // 16)
    pltpu.sync_copy(h8.at[pl.ds(nbase, NPAD // 16)],
                    acc.at[pl.ds(nbase, NPAD // 16)])
    plsc.subcore_barrier()
    _gather_phase(acc, dsti, srci, xi_out, xj_out,
                  idxd, idxs, rowsi, rowsj, semi, semj)


@functools.partial(
    pl.kernel,
    out_type=(jax.ShapeDtypeStruct((RP, 128, HPAD), jnp.float32),
              jax.ShapeDtypeStruct((RP, 128, HPAD), jnp.float32)),
    mesh=_MESH,
    compiler_params=pltpu.CompilerParams(use_tc_tiling_on_sc=False),
    scratch_types=[
        pltpu.VMEM_SHARED((NPAD, HPAD), jnp.float32),
        pltpu.VMEM((KB, 128), jnp.int32),
        pltpu.VMEM((KB, 128, HPAD), jnp.float32),
        pltpu.VMEM((KG, 128), jnp.int32),
        pltpu.VMEM((KG, 128), jnp.int32),
        pltpu.VMEM((KG, 128, HPAD), jnp.float32),
        pltpu.VMEM((KG, 128, HPAD), jnp.float32),
        pltpu.SemaphoreType.DMA,
        pltpu.SemaphoreType.DMA,
    ],
)
def _sc_scatter_gather(msg3, dsti, srci, zeros, xi_out, xj_out,
                       acc, idxb, rowsb, idxd, idxs, rowsi, rowsj,
                       semi, semj):
    _scatter_phase(acc, msg3, dsti, zeros, idxb, rowsb, semi)
    _gather_phase(acc, dsti, srci, xi_out, xj_out,
                  idxd, idxs, rowsi, rowsj, semi, semj)


@functools.partial(
    pl.kernel,
    out_type=(jax.ShapeDtypeStruct((NUM_GRAPHS, HPAD), jnp.float32),
              jax.ShapeDtypeStruct((NUM_GRAPHS, HPAD), jnp.float32)),
    mesh=_MESH,
    compiler_params=pltpu.CompilerParams(use_tc_tiling_on_sc=False),
    scratch_types=[
        pltpu.VMEM_SHARED((NPAD, HPAD), jnp.float32),
        pltpu.VMEM_SHARED((GPAD, HPAD), jnp.float32),
        pltpu.VMEM_SHARED((GPAD, HPAD), jnp.float32),
        pltpu.VMEM((KB, 128), jnp.int32),
        pltpu.VMEM((KB, 128, HPAD), jnp.float32),
        pltpu.VMEM((NR_W, 128), jnp.int32),
        pltpu.VMEM((NPAD // 16, HPAD), jnp.float32),
        pltpu.VMEM((NPAD // 16, HPAD), jnp.float32),
        pltpu.SemaphoreType.DMA,
        pltpu.SemaphoreType.DMA,
    ],
)
def _sc_scatter_pool(msg3, dsti, zeros, batchi, ones2, sums_out, cnt_out,
                     acc, accs, accc, idxb, rowsb, idxp, rowsp, onesb,
                     sema, semb):
    core = lax.axis_index("c")
    w = lax.axis_index("s")

    @pl.when(core == 0)
    def _():
        gb = w * (GPAD // 16)
        pltpu.sync_copy(zeros.at[pl.ds(gb, GPAD // 16)],
                        accs.at[pl.ds(gb, GPAD // 16)])
        pltpu.sync_copy(zeros.at[pl.ds(gb, GPAD // 16)],
                        accc.at[pl.ds(gb, GPAD // 16)])
        _scatter_phase(acc, msg3, dsti, zeros, idxb, rowsb, sema)
        row0 = w * NR_W
        nbase = w * (NPAD // 16)
        pltpu.sync_copy(batchi.at[pl.ds(row0, NR_W)], idxp)
        pltpu.sync_copy(acc.at[pl.ds(nbase, NPAD // 16)], rowsp)
        pltpu.sync_copy(ones2, onesb)

        def prow(g, carry):
            a = pltpu.async_copy(rowsp.at[pl.ds(g * 128, 128)],
                                 accs.at[idxp.at[g]], sema, add=True)
            b = pltpu.async_copy(onesb.at[pl.ds(g * 128, 128)],
                                 accc.at[idxp.at[g]], semb, add=True)
            a.wait()
            b.wait()
            return carry

        lax.fori_loop(0, NR_W, prow, 0)
        plsc.subcore_barrier()

        @pl.when(w == 0)
        def _():
            pltpu.sync_copy(accs.at[pl.ds(0, NUM_GRAPHS)], sums_out)
            pltpu.sync_copy(accc.at[pl.ds(0, NUM_GRAPHS)], cnt_out)


def _mlp_edge_body(xi_ref, xj_ref, ea_ref, t_ref, w0a_ref, w0b_ref, w0r_ref,
                   b0_ref, w1_ref, b1_ref, w2_ref, b2_ref, out_ref):
    xi = xi_ref[:, 0:3]
    xj = xj_ref[:, 0:3]
    ea = ea_ref[...]
    xjt = jnp.dot(xj, t_ref[...], preferred_element_type=jnp.float32)
    p = (ea * xjt).astype(jnp.bfloat16)
    z = (jnp.dot(xi.astype(jnp.bfloat16), w0a_ref[...],
                 preferred_element_type=jnp.float32)
         + jnp.dot(xj.astype(jnp.bfloat16), w0b_ref[...],
                   preferred_element_type=jnp.float32)
         + jnp.dot(p, w0r_ref[...], preferred_element_type=jnp.float32)
         + b0_ref[...])
    z = jnp.where(z > 0, z, jnp.exp(z) - 1.0)
    z = jnp.dot(z.astype(jnp.bfloat16), w1_ref[...],
                preferred_element_type=jnp.float32)
    z = z + b1_ref[...]
    z = jnp.where(z > 0, z, jnp.exp(z) - 1.0)
    z = jnp.dot(z.astype(jnp.bfloat16), w2_ref[...],
                preferred_element_type=jnp.float32)
    z = z + b2_ref[...]
    out_ref[...] = jnp.concatenate(
        [z, jnp.zeros((z.shape[0], HPAD - 3), jnp.float32)], axis=1)


def _mlp_edges(xi8, xj8, ea, w0, b0, w1, b1, w2, b2):
    tmat = jnp.tile(jnp.eye(3, dtype=jnp.float32), (1, 3))
    w0rep = jnp.repeat(w0[6:9], 3, axis=0)
    full = lambda shape: pl.BlockSpec(shape, lambda i: (0, 0))
    return pl.pallas_call(
        _mlp_edge_body,
        grid=(E // BE,),
        in_specs=[
            pl.BlockSpec((BE, HPAD), lambda i: (i, 0)),
            pl.BlockSpec((BE, HPAD), lambda i: (i, 0)),
            pl.BlockSpec((BE, 9), lambda i: (i, 0)),
            full((3, 9)),
            full((3, 128)), full((3, 128)), full((9, 128)), full((1, 128)),
            full((128, 128)), full((1, 128)),
            full((128, 3)), full((1, 3)),
        ],
        out_specs=pl.BlockSpec((BE, HPAD), lambda i: (i, 0)),
        out_shape=jax.ShapeDtypeStruct((EP, HPAD), jnp.float32),
    )(xi8, xj8, ea, tmat,
      w0[0:3].astype(jnp.bfloat16), w0[3:6].astype(jnp.bfloat16),
      w0rep.astype(jnp.bfloat16), b0.reshape(1, -1),
      w1.astype(jnp.bfloat16), b1.reshape(1, -1),
      w2.astype(jnp.bfloat16), b2.reshape(1, -1))


def _cls_body(s_ref, c_ref, w0_ref, b0_ref, w1_ref, b1_ref, w2_ref, b2_ref,
              out_ref):
    p = s_ref[:, 0:3] / jnp.maximum(c_ref[:, 0:1], 1.0)
    z = jnp.dot(p, w0_ref[...], preferred_element_type=jnp.float32) + b0_ref[...]
    z = jnp.where(z > 0, z, jnp.exp(z) - 1.0)
    z = jnp.dot(z, w1_ref[...], preferred_element_type=jnp.float32) + b1_ref[...]
    z = jnp.where(z > 0, z, jnp.exp(z) - 1.0)
    out_ref[...] = (jnp.dot(z, w2_ref[...], preferred_element_type=jnp.float32)
                    + b2_ref[...])


def _cls_mlp(sums, cnt, w0, b0, w1, b1, w2, b2):
    return pl.pallas_call(
        _cls_body,
        out_shape=jax.ShapeDtypeStruct((NUM_GRAPHS, 2), jnp.float32),
    )(sums, cnt, w0, b0.reshape(1, -1), w1, b1.reshape(1, -1),
      w2, b2.reshape(1, -1))


def kernel(x, edge_index, edge_attr, batch,
           mp0_W0, mp0_b0, mp0_W1, mp0_b1, mp0_W2, mp0_b2,
           mp1_W0, mp1_b0, mp1_W1, mp1_b1, mp1_W2, mp1_b2,
           mp2_W0, mp2_b0, mp2_W1, mp2_b1, mp2_W2, mp2_b2,
           cls_W0, cls_b0, cls_W1, cls_b1, cls_W2, cls_b2):
    src = edge_index[0]
    dst = edge_index[1]
    epad = jnp.full((EP - E,), N, jnp.int32)
    dsti = jnp.concatenate([dst, epad]).reshape(RP, 128)
    srci = jnp.concatenate([src, epad]).reshape(RP, 128)
    batchi = jnp.concatenate(
        [batch, jnp.full((NPAD - N,), NUM_GRAPHS, jnp.int32)]).reshape(NROWS, 128)
    zeros = jnp.zeros((NPAD, HPAD), jnp.float32)
    ones2 = jnp.ones((NPAD // 16, HPAD), jnp.float32)
    h8 = zeros.at[0:N, 0:3].set(x)

    mp = [(mp0_W0, mp0_b0, mp0_W1, mp0_b1, mp0_W2, mp0_b2),
          (mp1_W0, mp1_b0, mp1_W1, mp1_b1, mp1_W2, mp1_b2),
          (mp2_W0, mp2_b0, mp2_W1, mp2_b1, mp2_W2, mp2_b2)]

    xi3, xj3 = _sc_gather0(h8, dsti, srci)
    msg = None
    for li, (w0, b0, w1, b1, w2, b2) in enumerate(mp):
        msg = _mlp_edges(xi3.reshape(EP, HPAD), xj3.reshape(EP, HPAD),
                         edge_attr, w0, b0, w1, b1, w2, b2)
        if li < 2:
            xi3, xj3 = _sc_scatter_gather(msg.reshape(RP, 128, HPAD),
                                          dsti, srci, zeros)

    sums, cnt = _sc_scatter_pool(msg.reshape(RP, 128, HPAD), dsti, zeros,
                                 batchi, ones2)
    return _cls_mlp(sums, cnt, cls_W0, cls_b0, cls_W1, cls_b1, cls_W2, cls_b2)


# speedup vs baseline: 1.0565x; 1.0565x over previous
import functools

import jax
import jax.numpy as jnp
from jax import lax
from jax.experimental import pallas as pl
from jax.experimental.pallas import tpu as pltpu
from jax.experimental.pallas import tpu_sc as plsc

N = 50000
E = 1600000
NUM_GRAPHS = 512
HPAD = 8
BE = 8000
RP = 12544
EP = RP * 128
NPAD = 51200
NROWS = NPAD // 128
ROWS_G = RP // 32
ROWS_S = RP // 16
KG = 8
KB = 16
NR_W = NROWS // 16
GPAD = 640

_MESH = plsc.VectorSubcoreMesh(core_axis_name="c", subcore_axis_name="s")


def _gather_phase(acc, dsti, srci, xi_out, xj_out,
                  idxd, idxs, rowsi, rowsj, semi, semj):
    wid = lax.axis_index("s") * 2 + lax.axis_index("c")
    base = wid * ROWS_G

    def chunk(g, carry):
        row0 = base + g * KG
        pltpu.sync_copy(dsti.at[pl.ds(row0, KG)], idxd)
        pltpu.sync_copy(srci.at[pl.ds(row0, KG)], idxs)
        cps = []
        for b in range(KG):
            cps.append(pltpu.async_copy(acc.at[idxd.at[b]], rowsi.at[b], semi))
            cps.append(pltpu.async_copy(acc.at[idxs.at[b]], rowsj.at[b], semj))
        for c in cps:
            c.wait()
        pltpu.sync_copy(rowsi, xi_out.at[pl.ds(row0, KG)])
        pltpu.sync_copy(rowsj, xj_out.at[pl.ds(row0, KG)])
        return carry

    lax.fori_loop(0, ROWS_G // KG, chunk, 0)


def _scatter_phase(acc, msg3, dsti, zeros, idxb, rowsb, sem):
    w = lax.axis_index("s")
    nbase = w * (NPAD // 16)
    pltpu.sync_copy(zeros.at[pl.ds(nbase, NPAD // 16)],
                    acc.at[pl.ds(nbase, NPAD // 16)])
    plsc.subcore_barrier()

    def chunk(g, carry):
        row0 = w * ROWS_S + g * KB
        pltpu.sync_copy(dsti.at[pl.ds(row0, KB)], idxb)
        pltpu.sync_copy(msg3.at[pl.ds(row0, KB)], rowsb)
        cps = [pltpu.async_copy(rowsb.at[b], acc.at[idxb.at[b]], sem,
                                add=True)
               for b in range(KB)]
        for c in cps:
            c.wait()
        return carry

    lax.fori_loop(0, ROWS_S // KB, chunk, 0)
    plsc.subcore_barrier()


@functools.partial(
    pl.kernel,
    out_type=(jax.ShapeDtypeStruct((RP, 128, HPAD), jnp.float32),
              jax.ShapeDtypeStruct((RP, 128, HPAD), jnp.float32)),
    mesh=_MESH,
    compiler_params=pltpu.CompilerParams(use_tc_tiling_on_sc=False),
    scratch_types=[
        pltpu.VMEM_SHARED((NPAD, HPAD), jnp.float32),
        pltpu.VMEM((KG, 128), jnp.int32),
        pltpu.VMEM((KG, 128), jnp.int32),
        pltpu.VMEM((KG, 128, HPAD), jnp.float32),
        pltpu.VMEM((KG, 128, HPAD), jnp.float32),
        pltpu.SemaphoreType.DMA,
        pltpu.SemaphoreType.DMA,
    ],
)
def _sc_gather0(h8, dsti, srci, xi_out, xj_out,
                acc, idxd, idxs, rowsi, rowsj, semi, semj):
    w = lax.axis_index("s")
    nbase = w * (NPAD // 16)
    pltpu.sync_copy(h8.at[pl.ds(nbase, NPAD // 16)],
                    acc.at[pl.ds(nbase, NPAD // 16)])
    plsc.subcore_barrier()
    _gather_phase(acc, dsti, srci, xi_out, xj_out,
                  idxd, idxs, rowsi, rowsj, semi, semj)


@functools.partial(
    pl.kernel,
    out_type=(jax.ShapeDtypeStruct((RP, 128, HPAD), jnp.float32),
              jax.ShapeDtypeStruct((RP, 128, HPAD), jnp.float32)),
    mesh=_MESH,
    compiler_params=pltpu.CompilerParams(use_tc_tiling_on_sc=False),
    scratch_types=[
        pltpu.VMEM_SHARED((NPAD, HPAD), jnp.float32),
        pltpu.VMEM((KB, 128), jnp.int32),
        pltpu.VMEM((KB, 128, HPAD), jnp.float32),
        pltpu.VMEM((KG, 128), jnp.int32),
        pltpu.VMEM((KG, 128), jnp.int32),
        pltpu.VMEM((KG, 128, HPAD), jnp.float32),
        pltpu.VMEM((KG, 128, HPAD), jnp.float32),
        pltpu.SemaphoreType.DMA,
        pltpu.SemaphoreType.DMA,
    ],
)
def _sc_scatter_gather(msg3, dsti, srci, zeros, xi_out, xj_out,
                       acc, idxb, rowsb, idxd, idxs, rowsi, rowsj,
                       semi, semj):
    _scatter_phase(acc, msg3, dsti, zeros, idxb, rowsb, semi)
    _gather_phase(acc, dsti, srci, xi_out, xj_out,
                  idxd, idxs, rowsi, rowsj, semi, semj)


@functools.partial(
    pl.kernel,
    out_type=(jax.ShapeDtypeStruct((NUM_GRAPHS, HPAD), jnp.float32),
              jax.ShapeDtypeStruct((NUM_GRAPHS, HPAD), jnp.float32)),
    mesh=_MESH,
    compiler_params=pltpu.CompilerParams(use_tc_tiling_on_sc=False),
    scratch_types=[
        pltpu.VMEM_SHARED((NPAD, HPAD), jnp.float32),
        pltpu.VMEM_SHARED((GPAD, HPAD), jnp.float32),
        pltpu.VMEM_SHARED((GPAD, HPAD), jnp.float32),
        pltpu.VMEM((KB, 128), jnp.int32),
        pltpu.VMEM((KB, 128, HPAD), jnp.float32),
        pltpu.VMEM((NR_W, 128), jnp.int32),
        pltpu.VMEM((NPAD // 16, HPAD), jnp.float32),
        pltpu.VMEM((NPAD // 16, HPAD), jnp.float32),
        pltpu.SemaphoreType.DMA,
        pltpu.SemaphoreType.DMA,
    ],
)
def _sc_scatter_pool(msg3, dsti, zeros, batchi, ones2, sums_out, cnt_out,
                     acc, accs, accc, idxb, rowsb, idxp, rowsp, onesb,
                     sema, semb):
    core = lax.axis_index("c")
    w = lax.axis_index("s")

    @pl.when(core == 0)
    def _():
        gb = w * (GPAD // 16)
        pltpu.sync_copy(zeros.at[pl.ds(gb, GPAD // 16)],
                        accs.at[pl.ds(gb, GPAD // 16)])
        pltpu.sync_copy(zeros.at[pl.ds(gb, GPAD // 16)],
                        accc.at[pl.ds(gb, GPAD // 16)])
        _scatter_phase(acc, msg3, dsti, zeros, idxb, rowsb, sema)
        row0 = w * NR_W
        nbase = w * (NPAD // 16)
        pltpu.sync_copy(batchi.at[pl.ds(row0, NR_W)], idxp)
        pltpu.sync_copy(acc.at[pl.ds(nbase, NPAD // 16)], rowsp)
        pltpu.sync_copy(ones2, onesb)

        def prow(g, carry):
            a = pltpu.async_copy(rowsp.at[pl.ds(g * 128, 128)],
                                 accs.at[idxp.at[g]], sema, add=True)
            b = pltpu.async_copy(onesb.at[pl.ds(g * 128, 128)],
                                 accc.at[idxp.at[g]], semb, add=True)
            a.wait()
            b.wait()
            return carry

        lax.fori_loop(0, NR_W, prow, 0)
        plsc.subcore_barrier()

        @pl.when(w == 0)
        def _():
            pltpu.sync_copy(accs.at[pl.ds(0, NUM_GRAPHS)], sums_out)
            pltpu.sync_copy(accc.at[pl.ds(0, NUM_GRAPHS)], cnt_out)


def _mlp_edge_body(xi_ref, xj_ref, ea_ref, t_ref, w0a_ref, w0b_ref, w0r_ref,
                   b0_ref, w1_ref, b1_ref, w2_ref, b2_ref, out_ref):
    xi = xi_ref[:, 0:3]
    xj = xj_ref[:, 0:3]
    ea = ea_ref[...]
    xjt = jnp.dot(xj, t_ref[...], preferred_element_type=jnp.float32)
    p = ea * xjt
    z = (jnp.dot(xi, w0a_ref[...], preferred_element_type=jnp.float32)
         + jnp.dot(xj, w0b_ref[...], preferred_element_type=jnp.float32)
         + jnp.dot(p, w0r_ref[...], preferred_element_type=jnp.float32)
         + b0_ref[...])
    z = jnp.where(z > 0, z, jnp.exp(z) - 1.0)
    z = jnp.dot(z, w1_ref[...], preferred_element_type=jnp.float32)
    z = z + b1_ref[...]
    z = jnp.where(z > 0, z, jnp.exp(z) - 1.0)
    z = jnp.dot(z, w2_ref[...], preferred_element_type=jnp.float32)
    z = z + b2_ref[...]
    out_ref[...] = jnp.concatenate(
        [z, jnp.zeros((z.shape[0], HPAD - 3), jnp.float32)], axis=1)


def _mlp_edges(xi8, xj8, ea, w0, b0, w1, b1, w2, b2):
    tmat = jnp.tile(jnp.eye(3, dtype=jnp.float32), (1, 3))
    w0rep = jnp.repeat(w0[6:9], 3, axis=0)
    full = lambda shape: pl.BlockSpec(shape, lambda i: (0, 0))
    return pl.pallas_call(
        _mlp_edge_body,
        grid=(E // BE,),
        in_specs=[
            pl.BlockSpec((BE, HPAD), lambda i: (i, 0)),
            pl.BlockSpec((BE, HPAD), lambda i: (i, 0)),
            pl.BlockSpec((BE, 9), lambda i: (i, 0)),
            full((3, 9)),
            full((3, 128)), full((3, 128)), full((9, 128)), full((1, 128)),
            full((128, 128)), full((1, 128)),
            full((128, 3)), full((1, 3)),
        ],
        out_specs=pl.BlockSpec((BE, HPAD), lambda i: (i, 0)),
        out_shape=jax.ShapeDtypeStruct((EP, HPAD), jnp.float32),
    )(xi8, xj8, ea, tmat, w0[0:3], w0[3:6], w0rep, b0.reshape(1, -1),
      w1, b1.reshape(1, -1), w2, b2.reshape(1, -1))


def _cls_body(s_ref, c_ref, w0_ref, b0_ref, w1_ref, b1_ref, w2_ref, b2_ref,
              out_ref):
    p = s_ref[:, 0:3] / jnp.maximum(c_ref[:, 0:1], 1.0)
    z = jnp.dot(p, w0_ref[...], preferred_element_type=jnp.float32) + b0_ref[...]
    z = jnp.where(z > 0, z, jnp.exp(z) - 1.0)
    z = jnp.dot(z, w1_ref[...], preferred_element_type=jnp.float32) + b1_ref[...]
    z = jnp.where(z > 0, z, jnp.exp(z) - 1.0)
    out_ref[...] = (jnp.dot(z, w2_ref[...], preferred_element_type=jnp.float32)
                    + b2_ref[...])


def _cls_mlp(sums, cnt, w0, b0, w1, b1, w2, b2):
    return pl.pallas_call(
        _cls_body,
        out_shape=jax.ShapeDtypeStruct((NUM_GRAPHS, 2), jnp.float32),
    )(sums, cnt, w0, b0.reshape(1, -1), w1, b1.reshape(1, -1),
      w2, b2.reshape(1, -1))


def kernel(x, edge_index, edge_attr, batch,
           mp0_W0, mp0_b0, mp0_W1, mp0_b1, mp0_W2, mp0_b2,
           mp1_W0, mp1_b0, mp1_W1, mp1_b1, mp1_W2, mp1_b2,
           mp2_W0, mp2_b0, mp2_W1, mp2_b1, mp2_W2, mp2_b2,
           cls_W0, cls_b0, cls_W1, cls_b1, cls_W2, cls_b2):
    src = edge_index[0]
    dst = edge_index[1]
    epad = jnp.full((EP - E,), N, jnp.int32)
    dsti = jnp.concatenate([dst, epad]).reshape(RP, 128)
    srci = jnp.concatenate([src, epad]).reshape(RP, 128)
    batchi = jnp.concatenate(
        [batch, jnp.full((NPAD - N,), NUM_GRAPHS, jnp.int32)]).reshape(NROWS, 128)
    zeros = jnp.zeros((NPAD, HPAD), jnp.float32)
    ones2 = jnp.ones((NPAD // 16, HPAD), jnp.float32)
    h8 = zeros.at[0:N, 0:3].set(x)

    mp = [(mp0_W0, mp0_b0, mp0_W1, mp0_b1, mp0_W2, mp0_b2),
          (mp1_W0, mp1_b0, mp1_W1, mp1_b1, mp1_W2, mp1_b2),
          (mp2_W0, mp2_b0, mp2_W1, mp2_b1, mp2_W2, mp2_b2)]

    xi3, xj3 = _sc_gather0(h8, dsti, srci)
    msg = None
    for li, (w0, b0, w1, b1, w2, b2) in enumerate(mp):
        msg = _mlp_edges(xi3.reshape(EP, HPAD), xj3.reshape(EP, HPAD),
                         edge_attr, w0, b0, w1, b1, w2, b2)
        if li < 2:
            xi3, xj3 = _sc_scatter_gather(msg.reshape(RP, 128, HPAD),
                                          dsti, srci, zeros)

    sums, cnt = _sc_scatter_pool(msg.reshape(RP, 128, HPAD), dsti, zeros,
                                 batchi, ones2)
    return _cls_mlp(sums, cnt, cls_W0, cls_b0, cls_W1, cls_b1, cls_W2, cls_b2)


# speedup vs baseline: 1.0717x; 1.0144x over previous
import functools

import jax
import jax.numpy as jnp
from jax import lax
from jax.experimental import pallas as pl
from jax.experimental.pallas import tpu as pltpu
from jax.experimental.pallas import tpu_sc as plsc

N = 50000
E = 1600000
NUM_GRAPHS = 512
HPAD = 8
BE = 10000
RP = 12544
EP = RP * 128
NPAD = 51200
NROWS = NPAD // 128
ROWS_G = RP // 32
ROWS_S = RP // 16
KG = 14
KB = 16
NR_W = NROWS // 16
GPAD = 640

_MESH = plsc.VectorSubcoreMesh(core_axis_name="c", subcore_axis_name="s")


def _gather_phase(acc, dsti, srci, xi_out, xj_out,
                  idxd, idxs, rowsi, rowsj, semi, semj):
    wid = lax.axis_index("s") * 2 + lax.axis_index("c")
    base = wid * ROWS_G

    def chunk(g, carry):
        row0 = base + g * KG
        pltpu.sync_copy(dsti.at[pl.ds(row0, KG)], idxd)
        pltpu.sync_copy(srci.at[pl.ds(row0, KG)], idxs)
        cps = []
        for b in range(KG):
            cps.append(pltpu.async_copy(acc.at[idxd.at[b]], rowsi.at[b], semi))
            cps.append(pltpu.async_copy(acc.at[idxs.at[b]], rowsj.at[b], semj))
        for c in cps:
            c.wait()
        pltpu.sync_copy(rowsi, xi_out.at[pl.ds(row0, KG)])
        pltpu.sync_copy(rowsj, xj_out.at[pl.ds(row0, KG)])
        return carry

    lax.fori_loop(0, ROWS_G // KG, chunk, 0)


def _scatter_phase(acc, msg3, dsti, zeros, idxb, rowsb, sem):
    w = lax.axis_index("s")
    nbase = w * (NPAD // 16)
    pltpu.sync_copy(zeros.at[pl.ds(nbase, NPAD // 16)],
                    acc.at[pl.ds(nbase, NPAD // 16)])
    plsc.subcore_barrier()

    def chunk(g, carry):
        row0 = w * ROWS_S + g * KB
        pltpu.sync_copy(dsti.at[pl.ds(row0, KB)], idxb)
        pltpu.sync_copy(msg3.at[pl.ds(row0, KB)], rowsb)
        cps = [pltpu.async_copy(rowsb.at[b], acc.at[idxb.at[b]], sem,
                                add=True)
               for b in range(KB)]
        for c in cps:
            c.wait()
        return carry

    lax.fori_loop(0, ROWS_S // KB, chunk, 0)
    plsc.subcore_barrier()


@functools.partial(
    pl.kernel,
    out_type=(jax.ShapeDtypeStruct((RP, 128, HPAD), jnp.float32),
              jax.ShapeDtypeStruct((RP, 128, HPAD), jnp.float32)),
    mesh=_MESH,
    compiler_params=pltpu.CompilerParams(use_tc_tiling_on_sc=False),
    scratch_types=[
        pltpu.VMEM_SHARED((NPAD, HPAD), jnp.float32),
        pltpu.VMEM((KG, 128), jnp.int32),
        pltpu.VMEM((KG, 128), jnp.int32),
        pltpu.VMEM((KG, 128, HPAD), jnp.float32),
        pltpu.VMEM((KG, 128, HPAD), jnp.float32),
        pltpu.SemaphoreType.DMA,
        pltpu.SemaphoreType.DMA,
    ],
)
def _sc_gather0(h8, dsti, srci, xi_out, xj_out,
                acc, idxd, idxs, rowsi, rowsj, semi, semj):
    w = lax.axis_index("s")
    nbase = w * (NPAD // 16)
    pltpu.sync_copy(h8.at[pl.ds(nbase, NPAD // 16)],
                    acc.at[pl.ds(nbase, NPAD // 16)])
    plsc.subcore_barrier()
    _gather_phase(acc, dsti, srci, xi_out, xj_out,
                  idxd, idxs, rowsi, rowsj, semi, semj)


@functools.partial(
    pl.kernel,
    out_type=(jax.ShapeDtypeStruct((RP, 128, HPAD), jnp.float32),
              jax.ShapeDtypeStruct((RP, 128, HPAD), jnp.float32)),
    mesh=_MESH,
    compiler_params=pltpu.CompilerParams(use_tc_tiling_on_sc=False),
    scratch_types=[
        pltpu.VMEM_SHARED((NPAD, HPAD), jnp.float32),
        pltpu.VMEM((KB, 128), jnp.int32),
        pltpu.VMEM((KB, 128, HPAD), jnp.float32),
        pltpu.VMEM((KG, 128), jnp.int32),
        pltpu.VMEM((KG, 128), jnp.int32),
        pltpu.VMEM((KG, 128, HPAD), jnp.float32),
        pltpu.VMEM((KG, 128, HPAD), jnp.float32),
        pltpu.SemaphoreType.DMA,
        pltpu.SemaphoreType.DMA,
    ],
)
def _sc_scatter_gather(msg3, dsti, srci, zeros, xi_out, xj_out,
                       acc, idxb, rowsb, idxd, idxs, rowsi, rowsj,
                       semi, semj):
    _scatter_phase(acc, msg3, dsti, zeros, idxb, rowsb, semi)
    _gather_phase(acc, dsti, srci, xi_out, xj_out,
                  idxd, idxs, rowsi, rowsj, semi, semj)


@functools.partial(
    pl.kernel,
    out_type=(jax.ShapeDtypeStruct((NUM_GRAPHS, HPAD), jnp.float32),
              jax.ShapeDtypeStruct((NUM_GRAPHS, HPAD), jnp.float32)),
    mesh=_MESH,
    compiler_params=pltpu.CompilerParams(use_tc_tiling_on_sc=False),
    scratch_types=[
        pltpu.VMEM_SHARED((NPAD, HPAD), jnp.float32),
        pltpu.VMEM_SHARED((GPAD, HPAD), jnp.float32),
        pltpu.VMEM_SHARED((GPAD, HPAD), jnp.float32),
        pltpu.VMEM((KB, 128), jnp.int32),
        pltpu.VMEM((KB, 128, HPAD), jnp.float32),
        pltpu.VMEM((NR_W, 128), jnp.int32),
        pltpu.VMEM((NPAD // 16, HPAD), jnp.float32),
        pltpu.VMEM((NPAD // 16, HPAD), jnp.float32),
        pltpu.SemaphoreType.DMA,
        pltpu.SemaphoreType.DMA,
    ],
)
def _sc_scatter_pool(msg3, dsti, zeros, batchi, ones2, sums_out, cnt_out,
                     acc, accs, accc, idxb, rowsb, idxp, rowsp, onesb,
                     sema, semb):
    core = lax.axis_index("c")
    w = lax.axis_index("s")

    @pl.when(core == 0)
    def _():
        gb = w * (GPAD // 16)
        pltpu.sync_copy(zeros.at[pl.ds(gb, GPAD // 16)],
                        accs.at[pl.ds(gb, GPAD // 16)])
        pltpu.sync_copy(zeros.at[pl.ds(gb, GPAD // 16)],
                        accc.at[pl.ds(gb, GPAD // 16)])
        _scatter_phase(acc, msg3, dsti, zeros, idxb, rowsb, sema)
        row0 = w * NR_W
        nbase = w * (NPAD // 16)
        pltpu.sync_copy(batchi.at[pl.ds(row0, NR_W)], idxp)
        pltpu.sync_copy(acc.at[pl.ds(nbase, NPAD // 16)], rowsp)
        pltpu.sync_copy(ones2, onesb)

        def prow(g, carry):
            a = pltpu.async_copy(rowsp.at[pl.ds(g * 128, 128)],
                                 accs.at[idxp.at[g]], sema, add=True)
            b = pltpu.async_copy(onesb.at[pl.ds(g * 128, 128)],
                                 accc.at[idxp.at[g]], semb, add=True)
            a.wait()
            b.wait()
            return carry

        lax.fori_loop(0, NR_W, prow, 0)
        plsc.subcore_barrier()

        @pl.when(w == 0)
        def _():
            pltpu.sync_copy(accs.at[pl.ds(0, NUM_GRAPHS)], sums_out)
            pltpu.sync_copy(accc.at[pl.ds(0, NUM_GRAPHS)], cnt_out)


def _mlp_edge_body(xi_ref, xj_ref, ea_ref, t_ref, w0a_ref, w0b_ref, w0r_ref,
                   b0_ref, w1_ref, b1_ref, w2_ref, b2_ref, out_ref):
    xi = xi_ref[:, 0:3]
    xj = xj_ref[:, 0:3]
    ea = ea_ref[...]
    xjt = jnp.dot(xj, t_ref[...], preferred_element_type=jnp.float32)
    p = ea * xjt
    z = (jnp.dot(xi, w0a_ref[...], preferred_element_type=jnp.float32)
         + jnp.dot(xj, w0b_ref[...], preferred_element_type=jnp.float32)
         + jnp.dot(p, w0r_ref[...], preferred_element_type=jnp.float32)
         + b0_ref[...])
    z = jnp.where(z > 0, z, jnp.exp(z) - 1.0)
    z = jnp.dot(z, w1_ref[...], preferred_element_type=jnp.float32)
    z = z + b1_ref[...]
    z = jnp.where(z > 0, z, jnp.exp(z) - 1.0)
    z = jnp.dot(z, w2_ref[...], preferred_element_type=jnp.float32)
    z = z + b2_ref[...]
    out_ref[...] = jnp.concatenate(
        [z, jnp.zeros((z.shape[0], HPAD - 3), jnp.float32)], axis=1)


def _mlp_edges(xi8, xj8, ea, w0, b0, w1, b1, w2, b2):
    tmat = jnp.tile(jnp.eye(3, dtype=jnp.float32), (1, 3))
    w0rep = jnp.repeat(w0[6:9], 3, axis=0)
    full = lambda shape: pl.BlockSpec(shape, lambda i: (0, 0))
    return pl.pallas_call(
        _mlp_edge_body,
        grid=(E // BE,),
        in_specs=[
            pl.BlockSpec((BE, HPAD), lambda i: (i, 0)),
            pl.BlockSpec((BE, HPAD), lambda i: (i, 0)),
            pl.BlockSpec((BE, 9), lambda i: (i, 0)),
            full((3, 9)),
            full((3, 128)), full((3, 128)), full((9, 128)), full((1, 128)),
            full((128, 128)), full((1, 128)),
            full((128, 3)), full((1, 3)),
        ],
        out_specs=pl.BlockSpec((BE, HPAD), lambda i: (i, 0)),
        out_shape=jax.ShapeDtypeStruct((EP, HPAD), jnp.float32),
    )(xi8, xj8, ea, tmat, w0[0:3], w0[3:6], w0rep, b0.reshape(1, -1),
      w1, b1.reshape(1, -1), w2, b2.reshape(1, -1))


def _cls_body(s_ref, c_ref, w0_ref, b0_ref, w1_ref, b1_ref, w2_ref, b2_ref,
              out_ref):
    p = s_ref[:, 0:3] / jnp.maximum(c_ref[:, 0:1], 1.0)
    z = jnp.dot(p, w0_ref[...], preferred_element_type=jnp.float32) + b0_ref[...]
    z = jnp.where(z > 0, z, jnp.exp(z) - 1.0)
    z = jnp.dot(z, w1_ref[...], preferred_element_type=jnp.float32) + b1_ref[...]
    z = jnp.where(z > 0, z, jnp.exp(z) - 1.0)
    out_ref[...] = (jnp.dot(z, w2_ref[...], preferred_element_type=jnp.float32)
                    + b2_ref[...])


def _cls_mlp(sums, cnt, w0, b0, w1, b1, w2, b2):
    return pl.pallas_call(
        _cls_body,
        out_shape=jax.ShapeDtypeStruct((NUM_GRAPHS, 2), jnp.float32),
    )(sums, cnt, w0, b0.reshape(1, -1), w1, b1.reshape(1, -1),
      w2, b2.reshape(1, -1))


def kernel(x, edge_index, edge_attr, batch,
           mp0_W0, mp0_b0, mp0_W1, mp0_b1, mp0_W2, mp0_b2,
           mp1_W0, mp1_b0, mp1_W1, mp1_b1, mp1_W2, mp1_b2,
           mp2_W0, mp2_b0, mp2_W1, mp2_b1, mp2_W2, mp2_b2,
           cls_W0, cls_b0, cls_W1, cls_b1, cls_W2, cls_b2):
    src = edge_index[0]
    dst = edge_index[1]
    epad = jnp.full((EP - E,), N, jnp.int32)
    dsti = jnp.concatenate([dst, epad]).reshape(RP, 128)
    srci = jnp.concatenate([src, epad]).reshape(RP, 128)
    batchi = jnp.concatenate(
        [batch, jnp.full((NPAD - N,), NUM_GRAPHS, jnp.int32)]).reshape(NROWS, 128)
    zeros = jnp.zeros((NPAD, HPAD), jnp.float32)
    ones2 = jnp.ones((NPAD // 16, HPAD), jnp.float32)
    h8 = zeros.at[0:N, 0:3].set(x)

    mp = [(mp0_W0, mp0_b0, mp0_W1, mp0_b1, mp0_W2, mp0_b2),
          (mp1_W0, mp1_b0, mp1_W1, mp1_b1, mp1_W2, mp1_b2),
          (mp2_W0, mp2_b0, mp2_W1, mp2_b1, mp2_W2, mp2_b2)]

    xi3, xj3 = _sc_gather0(h8, dsti, srci)
    msg = None
    for li, (w0, b0, w1, b1, w2, b2) in enumerate(mp):
        msg = _mlp_edges(xi3.reshape(EP, HPAD), xj3.reshape(EP, HPAD),
                         edge_attr, w0, b0, w1, b1, w2, b2)
        if li < 2:
            xi3, xj3 = _sc_scatter_gather(msg.reshape(RP, 128, HPAD),
                                          dsti, srci, zeros)

    sums, cnt = _sc_scatter_pool(msg.reshape(RP, 128, HPAD), dsti, zeros,
                                 batchi, ones2)
    return _cls_mlp(sums, cnt, cls_W0, cls_b0, cls_W1, cls_b1, cls_W2, cls_b2)
